# baseline (speedup 1.0000x reference)
import functools

import jax
import jax.numpy as jnp
from jax import lax
from jax.experimental import pallas as pl
from jax.experimental.pallas import tpu as pltpu
from jax.experimental.pallas import tpu_sc as plsc

_N = 10000
_E = 320000
_IN = 128
_HID = 64
_OUT = 16
_K = 10

_NT = 16
_NP = 10240
_EP = 327680
_EPT = _EP // _NT
_C = 1024
_NCHUNK = _EPT // _C
_RPT = _NP // _NT


def _mlp_body(x_ref, w1_ref, b1_ref, w2_ref, b2_ref, o_ref):
    h = jnp.dot(x_ref[...], w1_ref[...], preferred_element_type=jnp.float32)
    h = jnp.maximum(h + b1_ref[...], 0.0)
    o_ref[...] = jnp.dot(h, w2_ref[...], preferred_element_type=jnp.float32) + b2_ref[...]


def _mlp(x, w1, b1, w2, b2):
    bm = 2000
    return pl.pallas_call(
        _mlp_body,
        grid=(_N // bm,),
        in_specs=[
            pl.BlockSpec((bm, _IN), lambda i: (i, 0)),
            pl.BlockSpec((_IN, _HID), lambda i: (0, 0)),
            pl.BlockSpec((1, _HID), lambda i: (0, 0)),
            pl.BlockSpec((_HID, _OUT), lambda i: (0, 0)),
            pl.BlockSpec((1, _OUT), lambda i: (0, 0)),
        ],
        out_specs=pl.BlockSpec((bm, _OUT), lambda i: (i, 0)),
        out_shape=jax.ShapeDtypeStruct((_N, _OUT), jnp.float32),
    )(x, w1, b1.reshape(1, _HID), w2, b2.reshape(1, _OUT))


def _rsqrt16(d):
    i = lax.bitcast_convert_type(d, jnp.int32)
    i = jnp.int32(0x5F3759DF) - (i >> 1)
    y = lax.bitcast_convert_type(i, jnp.float32)
    for _ in range(3):
        y = y * (1.5 - 0.5 * d * y * y)
    return y


@functools.partial(
    pl.kernel,
    out_type=(jax.ShapeDtypeStruct((_NP, _OUT), jnp.float32),
              jax.ShapeDtypeStruct((_NP, _OUT), jnp.float32)),
    mesh=plsc.VectorSubcoreMesh(core_axis_name="c", subcore_axis_name="s"),
    compiler_params=pltpu.CompilerParams(use_tc_tiling_on_sc=False),
    scratch_types=[
        pltpu.VMEM_SHARED((_NP, _OUT), jnp.float32),
        pltpu.VMEM((_NCHUNK, _C), jnp.int32),
        pltpu.VMEM((_NCHUNK, _C), jnp.int32),
        pltpu.VMEM((_C, _OUT), jnp.float32),
        pltpu.VMEM((_C, _OUT), jnp.float32),
        pltpu.VMEM((_RPT, _OUT), jnp.float32),
        pltpu.VMEM((_RPT, _OUT), jnp.float32),
        pltpu.VMEM((_RPT, _OUT), jnp.float32),
        pltpu.VMEM((16, 16), jnp.float32),
        pltpu.SemaphoreType.DMA,
        pltpu.SemaphoreType.DMA,
    ],
)
def _prop(row_hbm, col_hbm, h_hbm, tempb_hbm, out_hbm, scaled,
          buf, ridx, cidx, msg0, msg1, stage, dis, hid,
          tempv, sem0, sem1):
    cid = lax.axis_index("c")
    sid = lax.axis_index("s")
    on = cid == 0
    ebase = sid * _EPT
    nbase = sid * _RPT

    @pl.when(on)
    def _load():
        for j in range(_NCHUNK):
            pltpu.sync_copy(row_hbm.at[pl.ds(ebase + j * _C, _C)], ridx.at[j])
            pltpu.sync_copy(col_hbm.at[pl.ds(ebase + j * _C, _C)], cidx.at[j])
        pltpu.sync_copy(tempb_hbm, tempv)

        def fill(i, _):
            msg0[i] = jnp.full((16,), 1.0, jnp.float32)
            return 0
        lax.fori_loop(0, _C, fill, 0)
        pltpu.sync_copy(msg0.at[pl.ds(0, _RPT)], buf.at[pl.ds(nbase, _RPT)])

    plsc.subcore_barrier()

    @pl.when(on)
    def _deg():
        def body(j, _):
            pltpu.sync_copy(msg0, buf.at[cidx.at[j]], add=True)
            return 0
        lax.fori_loop(0, _NCHUNK, body, 0)

    plsc.subcore_barrier()

    @pl.when(on)
    def _init():
        pltpu.sync_copy(buf.at[pl.ds(nbase, _RPT)], stage)
        t0 = tempv[0]

        def body1(i, _):
            dis[i] = _rsqrt16(stage[i])
            return 0
        lax.fori_loop(0, _RPT, body1, 0)

        pltpu.sync_copy(h_hbm.at[pl.ds(nbase, _RPT)], stage)

        def body2(i, _):
            hv = stage[i]
            hid[i] = t0 * hv
            stage[i] = dis[i] * hv
            return 0
        lax.fori_loop(0, _RPT, body2, 0)
        pltpu.sync_copy(stage, scaled.at[pl.ds(nbase, _RPT)])
        pltpu.sync_copy(stage, buf.at[pl.ds(nbase, _RPT)])

    plsc.subcore_barrier()

    for k in range(_K):
        last = k == _K - 1

        @pl.when(on)
        def _scatter():
            pltpu.make_async_copy(scaled.at[ridx.at[0]], msg0, sem0).start()

            def body(i, _):
                j = 2 * i
                pltpu.make_async_copy(
                    scaled.at[ridx.at[j + 1]], msg1, sem1).start()
                pltpu.make_async_copy(scaled.at[ridx.at[j]], msg0, sem0).wait()
                pltpu.sync_copy(msg0, buf.at[cidx.at[j]], add=True)

                @pl.when(i < _NCHUNK // 2 - 1)
                def _():
                    pltpu.make_async_copy(
                        scaled.at[ridx.at[j + 2]], msg0, sem0).start()
                pltpu.make_async_copy(
                    scaled.at[ridx.at[j + 1]], msg1, sem1).wait()
                pltpu.sync_copy(msg1, buf.at[cidx.at[j + 1]], add=True)
                return 0
            lax.fori_loop(0, _NCHUNK // 2, body, 0)

        plsc.subcore_barrier()

        @pl.when(on)
        def _update(k=k, last=last):
            pltpu.sync_copy(buf.at[pl.ds(nbase, _RPT)], stage)
            tk = tempv[k + 1]

            def body(i, _):
                c = dis[i] * stage[i]
                hid[i] = hid[i] + tk * c
                if not last:
                    stage[i] = dis[i] * c
                return 0
            lax.fori_loop(0, _RPT, body, 0)
            if not last:
                pltpu.sync_copy(stage, scaled.at[pl.ds(nbase, _RPT)])
                pltpu.sync_copy(stage, buf.at[pl.ds(nbase, _RPT)])

        plsc.subcore_barrier()

    @pl.when(on)
    def _out():
        pltpu.sync_copy(hid, out_hbm.at[pl.ds(nbase, _RPT)])


def kernel(x, edge_index, w1, b1, w2, b2, temp):
    h = _mlp(x, w1, b1, w2, b2)
    pad = jnp.full((_EP - _E,), _N, jnp.int32)
    row = jnp.concatenate([edge_index[0], pad])
    col = jnp.concatenate([edge_index[1], pad])
    h_p = jnp.pad(h, ((0, _NP - _N), (0, 0)))
    tempb = jnp.zeros((16, 16), jnp.float32).at[: _K + 1, :].set(temp[:, None])
    out, _ = _prop(row, col, h_p, tempb)
    return out[:_N]


# speedup vs baseline: 1.8207x; 1.8207x over previous
import functools

import jax
import jax.numpy as jnp
from jax import lax
from jax.experimental import pallas as pl
from jax.experimental.pallas import tpu as pltpu
from jax.experimental.pallas import tpu_sc as plsc

_N = 10000
_E = 320000
_IN = 128
_HID = 64
_OUT = 16
_K = 10

_NT = 16
_NP = 10240
_EP = 327680
_EPT = _EP // _NT
_C = 1024
_NCHUNK = _EPT // _C
_RPT = _NP // _NT


def _mlp_body(x_ref, w1_ref, b1_ref, w2_ref, b2_ref, o_ref):
    h = jnp.dot(x_ref[...], w1_ref[...], preferred_element_type=jnp.float32)
    h = jnp.maximum(h + b1_ref[...], 0.0)
    o_ref[...] = jnp.dot(h, w2_ref[...], preferred_element_type=jnp.float32) + b2_ref[...]


def _mlp(x, w1, b1, w2, b2):
    bm = 2000
    return pl.pallas_call(
        _mlp_body,
        grid=(_N // bm,),
        in_specs=[
            pl.BlockSpec((bm, _IN), lambda i: (i, 0)),
            pl.BlockSpec((_IN, _HID), lambda i: (0, 0)),
            pl.BlockSpec((1, _HID), lambda i: (0, 0)),
            pl.BlockSpec((_HID, _OUT), lambda i: (0, 0)),
            pl.BlockSpec((1, _OUT), lambda i: (0, 0)),
        ],
        out_specs=pl.BlockSpec((bm, _OUT), lambda i: (i, 0)),
        out_shape=jax.ShapeDtypeStruct((_N, _OUT), jnp.float32),
    )(x, w1, b1.reshape(1, _HID), w2, b2.reshape(1, _OUT))


def _rsqrt16(d):
    i = lax.bitcast_convert_type(d, jnp.int32)
    i = jnp.int32(0x5F3759DF) - (i >> 1)
    y = lax.bitcast_convert_type(i, jnp.float32)
    for _ in range(3):
        y = y * (1.5 - 0.5 * d * y * y)
    return y


@functools.partial(
    pl.kernel,
    out_type=jax.ShapeDtypeStruct((_NP, _OUT), jnp.float32),
    mesh=plsc.VectorSubcoreMesh(core_axis_name="c", subcore_axis_name="s"),
    compiler_params=pltpu.CompilerParams(use_tc_tiling_on_sc=False),
    scratch_types=[
        pltpu.VMEM_SHARED((_NP, _OUT), jnp.float32),
        pltpu.VMEM_SHARED((_NP, _OUT), jnp.float32),
        pltpu.VMEM((_NCHUNK, _C), jnp.int32),
        pltpu.VMEM((_NCHUNK, _C), jnp.int32),
        pltpu.VMEM((_C, _OUT), jnp.float32),
        pltpu.VMEM((_C, _OUT), jnp.float32),
        pltpu.VMEM((_RPT, _OUT), jnp.float32),
        pltpu.VMEM((_RPT, _OUT), jnp.float32),
        pltpu.VMEM((_RPT, _OUT), jnp.float32),
        pltpu.VMEM((16, 16), jnp.float32),
        [pltpu.SemaphoreType.DMA] * 4,
    ],
)
def _prop(row_hbm, col_hbm, h_hbm, tempb_hbm, out_hbm,
          scaled, buf, ridx, cidx, msg0, msg1, stage, dis, hid,
          tempv, sems):
    sem0, sem1, sem2, sem3 = sems[0], sems[1], sems[2], sems[3]
    cid = lax.axis_index("c")
    sid = lax.axis_index("s")
    on = cid == 0
    ebase = sid * _EPT
    nbase = sid * _RPT

    @pl.when(on)
    def _load():
        for j in range(_NCHUNK):
            pltpu.sync_copy(row_hbm.at[pl.ds(ebase + j * _C, _C)], ridx.at[j])
            pltpu.sync_copy(col_hbm.at[pl.ds(ebase + j * _C, _C)], cidx.at[j])
        pltpu.sync_copy(tempb_hbm, tempv)

        def fill(i, _):
            msg0[i] = jnp.full((16,), 1.0, jnp.float32)
            return 0
        lax.fori_loop(0, _C, fill, 0)
        pltpu.sync_copy(msg0.at[pl.ds(0, _RPT)], buf.at[pl.ds(nbase, _RPT)])

    plsc.subcore_barrier()

    @pl.when(on)
    def _deg():
        def body(j, _):
            pltpu.sync_copy(msg0, buf.at[cidx.at[j]], add=True)
            return 0
        lax.fori_loop(0, _NCHUNK, body, 0)

    plsc.subcore_barrier()

    @pl.when(on)
    def _init():
        pltpu.sync_copy(buf.at[pl.ds(nbase, _RPT)], stage)
        t0 = tempv[0]

        def body1(i, _):
            dis[i] = _rsqrt16(stage[i])
            return 0
        lax.fori_loop(0, _RPT, body1, 0)

        pltpu.sync_copy(h_hbm.at[pl.ds(nbase, _RPT)], stage)

        def body2(i, _):
            hv = stage[i]
            hid[i] = t0 * hv
            stage[i] = dis[i] * hv
            return 0
        lax.fori_loop(0, _RPT, body2, 0)
        pltpu.sync_copy(stage, scaled.at[pl.ds(nbase, _RPT)])
        pltpu.sync_copy(stage, buf.at[pl.ds(nbase, _RPT)])

    plsc.subcore_barrier()

    for k in range(_K):
        last = k == _K - 1

        @pl.when(on)
        def _scatter():
            pltpu.make_async_copy(scaled.at[ridx.at[0]], msg0, sem0).start()

            def body(i, _):
                j = 2 * i
                pltpu.make_async_copy(
                    scaled.at[ridx.at[j + 1]], msg1, sem1).start()
                pltpu.make_async_copy(scaled.at[ridx.at[j]], msg0, sem0).wait()
                pltpu.sync_copy(msg0, buf.at[cidx.at[j]], add=True)

                @pl.when(i < _NCHUNK // 2 - 1)
                def _():
                    pltpu.make_async_copy(
                        scaled.at[ridx.at[j + 2]], msg0, sem0).start()
                pltpu.make_async_copy(
                    scaled.at[ridx.at[j + 1]], msg1, sem1).wait()
                pltpu.sync_copy(msg1, buf.at[cidx.at[j + 1]], add=True)
                return 0
            lax.fori_loop(0, _NCHUNK // 2, body, 0)

        plsc.subcore_barrier()

        @pl.when(on)
        def _update(k=k, last=last):
            hf = _RPT // 2
            pltpu.make_async_copy(
                buf.at[pl.ds(nbase, hf)], stage.at[pl.ds(0, hf)], sem0).start()
            pltpu.make_async_copy(
                buf.at[pl.ds(nbase + hf, hf)], stage.at[pl.ds(hf, hf)],
                sem1).start()
            tk = tempv[k + 1]

            def body(i, _):
                c = dis[i] * stage[i]
                hid[i] = hid[i] + tk * c
                if not last:
                    stage[i] = dis[i] * c
                return 0

            pltpu.make_async_copy(
                buf.at[pl.ds(nbase, hf)], stage.at[pl.ds(0, hf)], sem0).wait()
            lax.fori_loop(0, hf, body, 0)
            if not last:
                pltpu.make_async_copy(
                    stage.at[pl.ds(0, hf)], scaled.at[pl.ds(nbase, hf)],
                    sem2).start()
                pltpu.make_async_copy(
                    stage.at[pl.ds(0, hf)], buf.at[pl.ds(nbase, hf)],
                    sem3).start()
            pltpu.make_async_copy(
                buf.at[pl.ds(nbase + hf, hf)], stage.at[pl.ds(hf, hf)],
                sem1).wait()
            lax.fori_loop(hf, _RPT, body, 0)
            if not last:
                pltpu.make_async_copy(
                    stage.at[pl.ds(hf, hf)],
                    scaled.at[pl.ds(nbase + hf, hf)], sem0).start()
                pltpu.make_async_copy(
                    stage.at[pl.ds(hf, hf)], buf.at[pl.ds(nbase + hf, hf)],
                    sem1).start()
                pltpu.make_async_copy(
                    stage.at[pl.ds(0, hf)], scaled.at[pl.ds(nbase, hf)],
                    sem2).wait()
                pltpu.make_async_copy(
                    stage.at[pl.ds(0, hf)], buf.at[pl.ds(nbase, hf)],
                    sem3).wait()
                pltpu.make_async_copy(
                    stage.at[pl.ds(hf, hf)],
                    scaled.at[pl.ds(nbase + hf, hf)], sem0).wait()
                pltpu.make_async_copy(
                    stage.at[pl.ds(hf, hf)], buf.at[pl.ds(nbase + hf, hf)],
                    sem1).wait()

        plsc.subcore_barrier()

    @pl.when(on)
    def _out():
        pltpu.sync_copy(hid, out_hbm.at[pl.ds(nbase, _RPT)])


def kernel(x, edge_index, w1, b1, w2, b2, temp):
    h = _mlp(x, w1, b1, w2, b2)
    pad = jnp.full((_EP - _E,), _N, jnp.int32)
    row = jnp.concatenate([edge_index[0], pad])
    col = jnp.concatenate([edge_index[1], pad])
    h_p = jnp.pad(h, ((0, _NP - _N), (0, 0)))
    tempb = jnp.zeros((16, 16), jnp.float32).at[: _K + 1, :].set(temp[:, None])
    out = _prop(row, col, h_p, tempb)
    return out[:_N]


# speedup vs baseline: 1.9047x; 1.0461x over previous
import functools

import jax
import jax.numpy as jnp
from jax import lax
from jax.experimental import pallas as pl
from jax.experimental.pallas import tpu as pltpu
from jax.experimental.pallas import tpu_sc as plsc

_N = 10000
_E = 320000
_IN = 128
_HID = 64
_OUT = 16
_K = 10

_NT = 16
_NP = 10240
_EP = 327680
_EPT = _EP // _NT
_C = 1024
_NCHUNK = _EPT // _C
_RPT = _NP // _NT


def _mlp_body(x_ref, w1_ref, b1_ref, w2_ref, b2_ref, o_ref):
    h = jnp.dot(x_ref[...], w1_ref[...], preferred_element_type=jnp.float32)
    h = jnp.maximum(h + b1_ref[...], 0.0)
    o_ref[...] = jnp.dot(h, w2_ref[...], preferred_element_type=jnp.float32) + b2_ref[...]


def _mlp(x, w1, b1, w2, b2):
    bm = 2000
    return pl.pallas_call(
        _mlp_body,
        grid=(_N // bm,),
        in_specs=[
            pl.BlockSpec((bm, _IN), lambda i: (i, 0)),
            pl.BlockSpec((_IN, _HID), lambda i: (0, 0)),
            pl.BlockSpec((1, _HID), lambda i: (0, 0)),
            pl.BlockSpec((_HID, _OUT), lambda i: (0, 0)),
            pl.BlockSpec((1, _OUT), lambda i: (0, 0)),
        ],
        out_specs=pl.BlockSpec((bm, _OUT), lambda i: (i, 0)),
        out_shape=jax.ShapeDtypeStruct((_N, _OUT), jnp.float32),
    )(x, w1, b1.reshape(1, _HID), w2, b2.reshape(1, _OUT))


def _rsqrt16(d):
    i = lax.bitcast_convert_type(d, jnp.int32)
    i = jnp.int32(0x5F3759DF) - (i >> 1)
    y = lax.bitcast_convert_type(i, jnp.float32)
    for _ in range(3):
        y = y * (1.5 - 0.5 * d * y * y)
    return y


@functools.partial(
    pl.kernel,
    out_type=jax.ShapeDtypeStruct((_NP, _OUT), jnp.float32),
    mesh=plsc.VectorSubcoreMesh(core_axis_name="c", subcore_axis_name="s"),
    compiler_params=pltpu.CompilerParams(use_tc_tiling_on_sc=False),
    scratch_types=[
        pltpu.VMEM_SHARED((_NP, _OUT), jnp.float32),
        pltpu.VMEM_SHARED((_NP, _OUT), jnp.float32),
        pltpu.VMEM((_NCHUNK, _C), jnp.int32),
        pltpu.VMEM((_NCHUNK, _C), jnp.int32),
        pltpu.VMEM((_C, _OUT), jnp.float32),
        pltpu.VMEM((_C, _OUT), jnp.float32),
        pltpu.VMEM((_RPT, _OUT), jnp.float32),
        pltpu.VMEM((_RPT, _OUT), jnp.float32),
        pltpu.VMEM((_RPT, _OUT), jnp.float32),
        pltpu.VMEM((16, 16), jnp.float32),
        [pltpu.SemaphoreType.DMA] * 4,
    ],
)
def _prop(row_hbm, col_hbm, h_hbm, tempb_hbm, out_hbm,
          scaled, buf, ridx, cidx, msg0, msg1, stage, dis, hid,
          tempv, sems):
    sem0, sem1, sem2, sem3 = sems[0], sems[1], sems[2], sems[3]
    cid = lax.axis_index("c")
    sid = lax.axis_index("s")
    on = cid == 0
    ebase = sid * _EPT
    nbase = sid * _RPT

    @pl.when(on)
    def _load():
        for j in range(_NCHUNK):
            pltpu.make_async_copy(
                row_hbm.at[pl.ds(ebase + j * _C, _C)], ridx.at[j], sem0).start()
            pltpu.make_async_copy(
                col_hbm.at[pl.ds(ebase + j * _C, _C)], cidx.at[j], sem1).start()
        pltpu.make_async_copy(
            h_hbm.at[pl.ds(nbase, _RPT)], msg1.at[pl.ds(0, _RPT)], sem2).start()
        pltpu.sync_copy(tempb_hbm, tempv)

        def fill(i, _):
            msg0[i] = jnp.full((16,), 1.0, jnp.float32)
            return 0
        lax.fori_loop(0, _C, fill, 0)
        pltpu.sync_copy(msg0.at[pl.ds(0, _RPT)], buf.at[pl.ds(nbase, _RPT)])
        for j in range(_NCHUNK):
            pltpu.make_async_copy(
                row_hbm.at[pl.ds(ebase + j * _C, _C)], ridx.at[j], sem0).wait()
            pltpu.make_async_copy(
                col_hbm.at[pl.ds(ebase + j * _C, _C)], cidx.at[j], sem1).wait()
        pltpu.make_async_copy(
            h_hbm.at[pl.ds(nbase, _RPT)], msg1.at[pl.ds(0, _RPT)], sem2).wait()

    plsc.subcore_barrier()

    @pl.when(on)
    def _deg():
        def body(j, _):
            pltpu.sync_copy(msg0, buf.at[cidx.at[j]], add=True)
            return 0
        lax.fori_loop(0, _NCHUNK, body, 0)

    plsc.subcore_barrier()

    @pl.when(on)
    def _init():
        pltpu.sync_copy(buf.at[pl.ds(nbase, _RPT)], stage)
        t0 = tempv[0]

        def body1(i, _):
            dis[i] = _rsqrt16(stage[i])
            return 0
        lax.fori_loop(0, _RPT, body1, 0)

        def body2(i, _):
            hv = msg1[i]
            hid[i] = t0 * hv
            stage[i] = dis[i] * hv
            return 0
        lax.fori_loop(0, _RPT, body2, 0)
        pltpu.sync_copy(stage, scaled.at[pl.ds(nbase, _RPT)])
        pltpu.sync_copy(stage, buf.at[pl.ds(nbase, _RPT)])

    plsc.subcore_barrier()

    for k in range(_K):
        last = k == _K - 1

        @pl.when(on)
        def _scatter():
            pltpu.make_async_copy(scaled.at[ridx.at[0]], msg0, sem0).start()

            def body(i, _):
                j = 2 * i
                pltpu.make_async_copy(
                    scaled.at[ridx.at[j + 1]], msg1, sem1).start()
                pltpu.make_async_copy(scaled.at[ridx.at[j]], msg0, sem0).wait()
                pltpu.sync_copy(msg0, buf.at[cidx.at[j]], add=True)

                @pl.when(i < _NCHUNK // 2 - 1)
                def _():
                    pltpu.make_async_copy(
                        scaled.at[ridx.at[j + 2]], msg0, sem0).start()
                pltpu.make_async_copy(
                    scaled.at[ridx.at[j + 1]], msg1, sem1).wait()
                pltpu.sync_copy(msg1, buf.at[cidx.at[j + 1]], add=True)
                return 0
            lax.fori_loop(0, _NCHUNK // 2, body, 0)

        plsc.subcore_barrier()

        @pl.when(on)
        def _update(k=k, last=last):
            hf = _RPT // 2
            pltpu.make_async_copy(
                buf.at[pl.ds(nbase, hf)], stage.at[pl.ds(0, hf)], sem0).start()
            pltpu.make_async_copy(
                buf.at[pl.ds(nbase + hf, hf)], stage.at[pl.ds(hf, hf)],
                sem1).start()
            tk = tempv[k + 1]

            def body(i, _):
                c = dis[i] * stage[i]
                hid[i] = hid[i] + tk * c
                if not last:
                    stage[i] = dis[i] * c
                return 0

            pltpu.make_async_copy(
                buf.at[pl.ds(nbase, hf)], stage.at[pl.ds(0, hf)], sem0).wait()
            lax.fori_loop(0, hf, body, 0)
            if not last:
                pltpu.make_async_copy(
                    stage.at[pl.ds(0, hf)], scaled.at[pl.ds(nbase, hf)],
                    sem2).start()
                pltpu.make_async_copy(
                    stage.at[pl.ds(0, hf)], buf.at[pl.ds(nbase, hf)],
                    sem3).start()
            pltpu.make_async_copy(
                buf.at[pl.ds(nbase + hf, hf)], stage.at[pl.ds(hf, hf)],
                sem1).wait()
            lax.fori_loop(hf, _RPT, body, 0)
            if not last:
                pltpu.make_async_copy(
                    stage.at[pl.ds(hf, hf)],
                    scaled.at[pl.ds(nbase + hf, hf)], sem0).start()
                pltpu.make_async_copy(
                    stage.at[pl.ds(hf, hf)], buf.at[pl.ds(nbase + hf, hf)],
                    sem1).start()
                pltpu.make_async_copy(
                    stage.at[pl.ds(0, hf)], scaled.at[pl.ds(nbase, hf)],
                    sem2).wait()
                pltpu.make_async_copy(
                    stage.at[pl.ds(0, hf)], buf.at[pl.ds(nbase, hf)],
                    sem3).wait()
                pltpu.make_async_copy(
                    stage.at[pl.ds(hf, hf)],
                    scaled.at[pl.ds(nbase + hf, hf)], sem0).wait()
                pltpu.make_async_copy(
                    stage.at[pl.ds(hf, hf)], buf.at[pl.ds(nbase + hf, hf)],
                    sem1).wait()

        plsc.subcore_barrier()

    @pl.when(on)
    def _out():
        pltpu.sync_copy(hid, out_hbm.at[pl.ds(nbase, _RPT)])


def kernel(x, edge_index, w1, b1, w2, b2, temp):
    h = _mlp(x, w1, b1, w2, b2)
    pad = jnp.full((_EP - _E,), _N, jnp.int32)
    row = jnp.concatenate([edge_index[0], pad])
    col = jnp.concatenate([edge_index[1], pad])
    h_p = jnp.pad(h, ((0, _NP - _N), (0, 0)))
    tempb = jnp.zeros((16, 16), jnp.float32).at[: _K + 1, :].set(temp[:, None])
    out = _prop(row, col, h_p, tempb)
    return out[:_N]


# speedup vs baseline: 2.2020x; 1.1561x over previous
import functools

import jax
import jax.numpy as jnp
from jax import lax
from jax.experimental import pallas as pl
from jax.experimental.pallas import tpu as pltpu
from jax.experimental.pallas import tpu_sc as plsc

_N = 10000
_E = 320000
_IN = 128
_HID = 64
_OUT = 16
_K = 10

_NT = 16
_NP = 10240
_EP = 327680
_EPT = _EP // _NT
_C = 1024
_NCHUNK = _EPT // _C
_RPT = _NP // _NT


def _mlp_body(x_ref, w1_ref, b1_ref, w2_ref, b2_ref, o_ref):
    h = jnp.dot(x_ref[...], w1_ref[...], preferred_element_type=jnp.float32)
    h = jnp.maximum(h + b1_ref[...], 0.0)
    o_ref[...] = jnp.dot(h, w2_ref[...], preferred_element_type=jnp.float32) + b2_ref[...]


def _mlp(x, w1, b1, w2, b2):
    bm = 2000
    return pl.pallas_call(
        _mlp_body,
        grid=(_N // bm,),
        in_specs=[
            pl.BlockSpec((bm, _IN), lambda i: (i, 0)),
            pl.BlockSpec((_IN, _HID), lambda i: (0, 0)),
            pl.BlockSpec((1, _HID), lambda i: (0, 0)),
            pl.BlockSpec((_HID, _OUT), lambda i: (0, 0)),
            pl.BlockSpec((1, _OUT), lambda i: (0, 0)),
        ],
        out_specs=pl.BlockSpec((bm, _OUT), lambda i: (i, 0)),
        out_shape=jax.ShapeDtypeStruct((_N, _OUT), jnp.float32),
    )(x, w1, b1.reshape(1, _HID), w2, b2.reshape(1, _OUT))


def _rsqrt16(d):
    i = lax.bitcast_convert_type(d, jnp.int32)
    i = jnp.int32(0x5F3759DF) - (i >> 1)
    y = lax.bitcast_convert_type(i, jnp.float32)
    for _ in range(3):
        y = y * (1.5 - 0.5 * d * y * y)
    return y


@functools.partial(
    pl.kernel,
    out_type=jax.ShapeDtypeStruct((_NP, _OUT), jnp.float32),
    mesh=plsc.VectorSubcoreMesh(core_axis_name="c", subcore_axis_name="s"),
    compiler_params=pltpu.CompilerParams(use_tc_tiling_on_sc=False),
    scratch_types=[
        pltpu.VMEM_SHARED((_NP, _OUT), jnp.float32),
        pltpu.VMEM_SHARED((_NP, _OUT), jnp.float32),
        pltpu.VMEM((_NCHUNK, _C), jnp.int32),
        pltpu.VMEM((_NCHUNK, _C), jnp.int32),
        pltpu.VMEM((_C, _OUT), jnp.float32),
        pltpu.VMEM((_C, _OUT), jnp.float32),
        pltpu.VMEM((_RPT, _OUT), jnp.float32),
        pltpu.VMEM((_RPT, _OUT), jnp.float32),
        pltpu.VMEM((_RPT, _OUT), jnp.float32),
        pltpu.VMEM((16, 16), jnp.float32),
        [pltpu.SemaphoreType.DMA] * 4,
    ],
)
def _prop(row_hbm, col_hbm, h_hbm, tempb_hbm, out_hbm,
          scaled, buf, ridx, cidx, msg0, msg1, stage, dis, hid,
          tempv, sems):
    sem0, sem1, sem2, sem3 = sems[0], sems[1], sems[2], sems[3]
    cid = lax.axis_index("c")
    sid = lax.axis_index("s")
    on = cid == 0
    ebase = sid * _EPT
    nbase = sid * _RPT

    @pl.when(on)
    def _load():
        for j in range(_NCHUNK):
            pltpu.make_async_copy(
                row_hbm.at[pl.ds(ebase + j * _C, _C)], ridx.at[j], sem0).start()
            pltpu.make_async_copy(
                col_hbm.at[pl.ds(ebase + j * _C, _C)], cidx.at[j], sem1).start()
        pltpu.make_async_copy(
            h_hbm.at[pl.ds(nbase, _RPT)], msg1.at[pl.ds(0, _RPT)], sem2).start()
        pltpu.sync_copy(tempb_hbm, tempv)

        def fill(i, _):
            msg0[i] = jnp.full((16,), 1.0, jnp.float32)
            return 0
        lax.fori_loop(0, _C, fill, 0)
        pltpu.sync_copy(msg0.at[pl.ds(0, _RPT)], buf.at[pl.ds(nbase, _RPT)])
        for j in range(_NCHUNK):
            pltpu.make_async_copy(
                row_hbm.at[pl.ds(ebase + j * _C, _C)], ridx.at[j], sem0).wait()
            pltpu.make_async_copy(
                col_hbm.at[pl.ds(ebase + j * _C, _C)], cidx.at[j], sem1).wait()
        pltpu.make_async_copy(
            h_hbm.at[pl.ds(nbase, _RPT)], msg1.at[pl.ds(0, _RPT)], sem2).wait()

    plsc.subcore_barrier()

    @pl.when(on)
    def _deg():
        def body(j, _):
            pltpu.sync_copy(msg0, buf.at[cidx.at[j]], add=True)
            return 0
        lax.fori_loop(0, _NCHUNK, body, 0)

    plsc.subcore_barrier()

    @pl.when(on)
    def _init():
        pltpu.sync_copy(buf.at[pl.ds(nbase, _RPT)], stage)
        t0 = tempv[0]

        def body1(i, _):
            dis[i] = _rsqrt16(stage[i])
            return 0
        lax.fori_loop(0, _RPT, body1, 0)

        def body2(i, _):
            hv = msg1[i]
            hid[i] = t0 * hv
            stage[i] = dis[i] * hv
            return 0
        lax.fori_loop(0, _RPT, body2, 0)
        pltpu.sync_copy(stage, scaled.at[pl.ds(nbase, _RPT)])
        pltpu.sync_copy(stage, buf.at[pl.ds(nbase, _RPT)])

    plsc.subcore_barrier()

    for k in range(_K):
        last = k == _K - 1

        @pl.when(on)
        def _scatter():
            pltpu.make_async_copy(scaled.at[ridx.at[0]], msg0, sem0).start()

            def body(i, _):
                j = 2 * i
                pltpu.make_async_copy(
                    scaled.at[ridx.at[j + 1]], msg1, sem1).start()
                pltpu.make_async_copy(scaled.at[ridx.at[j]], msg0, sem0).wait()
                pltpu.sync_copy(msg0, buf.at[cidx.at[j]], add=True)

                @pl.when(i < _NCHUNK // 2 - 1)
                def _():
                    pltpu.make_async_copy(
                        scaled.at[ridx.at[j + 2]], msg0, sem0).start()
                pltpu.make_async_copy(
                    scaled.at[ridx.at[j + 1]], msg1, sem1).wait()
                pltpu.sync_copy(msg1, buf.at[cidx.at[j + 1]], add=True)
                return 0
            lax.fori_loop(0, _NCHUNK // 2, body, 0)

        plsc.subcore_barrier()

        @pl.when(on)
        def _update(k=k, last=last):
            hf = _RPT // 2
            pltpu.make_async_copy(
                buf.at[pl.ds(nbase, hf)], stage.at[pl.ds(0, hf)], sem0).start()
            pltpu.make_async_copy(
                buf.at[pl.ds(nbase + hf, hf)], stage.at[pl.ds(hf, hf)],
                sem1).start()
            tk = tempv[k + 1]

            def body(i, _):
                c = dis[i] * stage[i]
                hid[i] = hid[i] + tk * c
                if not last:
                    stage[i] = dis[i] * c
                return 0

            pltpu.make_async_copy(
                buf.at[pl.ds(nbase, hf)], stage.at[pl.ds(0, hf)], sem0).wait()
            lax.fori_loop(0, hf, body, 0)
            if not last:
                pltpu.make_async_copy(
                    stage.at[pl.ds(0, hf)], scaled.at[pl.ds(nbase, hf)],
                    sem2).start()
                pltpu.make_async_copy(
                    stage.at[pl.ds(0, hf)], buf.at[pl.ds(nbase, hf)],
                    sem3).start()
            pltpu.make_async_copy(
                buf.at[pl.ds(nbase + hf, hf)], stage.at[pl.ds(hf, hf)],
                sem1).wait()
            lax.fori_loop(hf, _RPT, body, 0)
            if not last:
                pltpu.make_async_copy(
                    stage.at[pl.ds(hf, hf)],
                    scaled.at[pl.ds(nbase + hf, hf)], sem0).start()
                pltpu.make_async_copy(
                    stage.at[pl.ds(hf, hf)], buf.at[pl.ds(nbase + hf, hf)],
                    sem1).start()
                pltpu.make_async_copy(
                    stage.at[pl.ds(0, hf)], scaled.at[pl.ds(nbase, hf)],
                    sem2).wait()
                pltpu.make_async_copy(
                    stage.at[pl.ds(0, hf)], buf.at[pl.ds(nbase, hf)],
                    sem3).wait()
                pltpu.make_async_copy(
                    stage.at[pl.ds(hf, hf)],
                    scaled.at[pl.ds(nbase + hf, hf)], sem0).wait()
                pltpu.make_async_copy(
                    stage.at[pl.ds(hf, hf)], buf.at[pl.ds(nbase + hf, hf)],
                    sem1).wait()

        plsc.subcore_barrier()

    @pl.when(on)
    def _out():
        pltpu.sync_copy(hid, out_hbm.at[pl.ds(nbase, _RPT)])


def kernel(x, edge_index, w1, b1, w2, b2, temp):
    h = _mlp(x, w1, b1, w2, b2)
    pad = _N + jnp.arange(_EP - _E, dtype=jnp.int32) % (_NP - _N)
    row = jnp.concatenate([edge_index[0], pad])
    col = jnp.concatenate([edge_index[1], pad])
    h_p = jnp.pad(h, ((0, _NP - _N), (0, 0)))
    tempb = jnp.zeros((16, 16), jnp.float32).at[: _K + 1, :].set(temp[:, None])
    out = _prop(row, col, h_p, tempb)
    return out[:_N]
